# Initial kernel scaffold; baseline (speedup 1.0000x reference)
import functools

import jax
import jax.numpy as jnp
from jax import lax
from jax.experimental import pallas as pl
from jax.experimental.pallas import tpu as pltpu
from jax.experimental.pallas import tpu_sc as plsc

BATCH = 16384
HIST = 200
HIDDEN = 32
TOTAL = BATCH * HIST

NUM_CORES = 2
NUM_SUBCORES = 16
NW = NUM_CORES * NUM_SUBCORES
PER_W = TOTAL // NW
CHUNK = 512
NCHUNK = PER_W // CHUNK

_mesh = plsc.VectorSubcoreMesh(core_axis_name="c", subcore_axis_name="s")


@functools.partial(
    pl.kernel,
    out_type=jax.ShapeDtypeStruct((TOTAL, HIDDEN), jnp.float32),
    mesh=_mesh,
    scratch_types=[
        pltpu.VMEM((CHUNK,), jnp.int32),
        pltpu.VMEM((CHUNK, HIDDEN), jnp.float32),
        pltpu.SemaphoreType.DMA,
    ],
)
def _emb_lookup(x_hbm, tab_hbm, out_hbm, idx_v, rows_v, gsem):
    wid = lax.axis_index("s") * NUM_CORES + lax.axis_index("c")
    base = wid * PER_W

    def body(j, carry):
        off = base + j * CHUNK
        pltpu.sync_copy(x_hbm.at[pl.ds(off, CHUNK)], idx_v)
        pltpu.async_copy(tab_hbm.at[idx_v], rows_v, gsem).wait()
        pltpu.sync_copy(rows_v, out_hbm.at[pl.ds(off, CHUNK)])
        return carry

    lax.fori_loop(0, NCHUNK, body, 0)


def kernel(x, table):
    flat = x.reshape(TOTAL)
    out = _emb_lookup(flat, table)
    return out.reshape(BATCH, HIST, HIDDEN)


# speedup vs baseline: 4.5605x; 4.5605x over previous
import functools

import jax
import jax.numpy as jnp
from jax import lax
from jax.experimental import pallas as pl
from jax.experimental.pallas import tpu as pltpu
from jax.experimental.pallas import tpu_sc as plsc

BATCH = 16384
HIST = 200
HIDDEN = 32
TOTAL = BATCH * HIST

NUM_CORES = 2
NUM_SUBCORES = 16
NW = NUM_CORES * NUM_SUBCORES
PER_W = TOTAL // NW
CHUNK = 512
NCHUNK = PER_W // CHUNK

_mesh = plsc.VectorSubcoreMesh(core_axis_name="c", subcore_axis_name="s")


@functools.partial(
    pl.kernel,
    out_type=jax.ShapeDtypeStruct((TOTAL, HIDDEN), jnp.float32),
    mesh=_mesh,
    scratch_types=[
        pltpu.VMEM((CHUNK,), jnp.int32),
        pltpu.VMEM((CHUNK, HIDDEN), jnp.float32),
        pltpu.SemaphoreType.DMA,
    ],
    compiler_params=pltpu.CompilerParams(use_tc_tiling_on_sc=False),
)
def _emb_lookup(x_hbm, tab_hbm, out_hbm, idx_v, rows_v, gsem):
    wid = lax.axis_index("s") * NUM_CORES + lax.axis_index("c")
    base = wid * PER_W

    def body(j, carry):
        off = base + j * CHUNK
        pltpu.sync_copy(x_hbm.at[pl.ds(off, CHUNK)], idx_v)
        pltpu.async_copy(tab_hbm.at[idx_v], rows_v, gsem).wait()
        pltpu.sync_copy(rows_v, out_hbm.at[pl.ds(off, CHUNK)])
        return carry

    lax.fori_loop(0, NCHUNK, body, 0)


def kernel(x, table):
    flat = x.reshape(TOTAL)
    out = _emb_lookup(flat, table)
    return out.reshape(BATCH, HIST, HIDDEN)


# speedup vs baseline: 5.0326x; 1.1035x over previous
import functools

import jax
import jax.numpy as jnp
from jax import lax
from jax.experimental import pallas as pl
from jax.experimental.pallas import tpu as pltpu
from jax.experimental.pallas import tpu_sc as plsc

BATCH = 16384
HIST = 200
HIDDEN = 32
TOTAL = BATCH * HIST

NUM_CORES = 2
NUM_SUBCORES = 16
NW = NUM_CORES * NUM_SUBCORES
PER_W = TOTAL // NW
CHUNK = 1024
NCHUNK = PER_W // CHUNK
NPAIR = NCHUNK // 2

_mesh = plsc.VectorSubcoreMesh(core_axis_name="c", subcore_axis_name="s")


@functools.partial(
    pl.kernel,
    out_type=jax.ShapeDtypeStruct((TOTAL, HIDDEN), jnp.float32),
    mesh=_mesh,
    scratch_types=[
        pltpu.VMEM((2, CHUNK), jnp.int32),
        pltpu.VMEM((2, CHUNK, HIDDEN), jnp.float32),
        pltpu.SemaphoreType.DMA((2,)),
        pltpu.SemaphoreType.DMA((2,)),
        pltpu.SemaphoreType.DMA((2,)),
    ],
    compiler_params=pltpu.CompilerParams(use_tc_tiling_on_sc=False),
)
def _emb_lookup(x_hbm, tab_hbm, out_hbm, idx_v, rows_v, isem, gsem, osem):
    wid = lax.axis_index("s") * NUM_CORES + lax.axis_index("c")
    base = wid * PER_W

    def xs(j):
        return x_hbm.at[pl.ds(base + j * CHUNK, CHUNK)]

    def outs(j):
        return out_hbm.at[pl.ds(base + j * CHUNK, CHUNK)]

    def istart(j, b):
        pltpu.async_copy(xs(j), idx_v.at[b], isem.at[b])

    def iwait(j, b):
        pltpu.make_async_copy(xs(j), idx_v.at[b], isem.at[b]).wait()

    def gstart(b):
        pltpu.async_copy(tab_hbm.at[idx_v.at[b]], rows_v.at[b], gsem.at[b])

    def gwait(b):
        pltpu.make_async_copy(tab_hbm.at[idx_v.at[b]], rows_v.at[b],
                              gsem.at[b]).wait()

    def ostart(j, b):
        pltpu.async_copy(rows_v.at[b], outs(j), osem.at[b])

    def owait(j, b):
        pltpu.make_async_copy(rows_v.at[b], outs(j), osem.at[b]).wait()

    def process(j, b, wait_prev_out, issue_next_idx):
        iwait(j, b)
        if wait_prev_out:
            owait(j - 2, b)
        gstart(b)
        gwait(b)
        ostart(j, b)
        if issue_next_idx:
            istart(j + 2, b)

    istart(0, 0)
    istart(1, 1)
    process(0, 0, False, True)
    process(1, 1, False, True)

    def body(p, carry):
        process(2 * p, 0, True, True)
        process(2 * p + 1, 1, True, True)
        return carry

    lax.fori_loop(1, NPAIR - 1, body, 0)

    process(NCHUNK - 2, 0, True, False)
    process(NCHUNK - 1, 1, True, False)
    owait(NCHUNK - 2, 0)
    owait(NCHUNK - 1, 1)


def kernel(x, table):
    flat = x.reshape(TOTAL)
    out = _emb_lookup(flat, table)
    return out.reshape(BATCH, HIST, HIDDEN)


# speedup vs baseline: 5.0493x; 1.0033x over previous
import functools

import jax
import jax.numpy as jnp
from jax import lax
from jax.experimental import pallas as pl
from jax.experimental.pallas import tpu as pltpu
from jax.experimental.pallas import tpu_sc as plsc

BATCH = 16384
HIST = 200
HIDDEN = 32
TOTAL = BATCH * HIST

NUM_CORES = 2
NUM_SUBCORES = 16
NW = NUM_CORES * NUM_SUBCORES
PER_W = TOTAL // NW
CHUNK = 512
NCHUNK = PER_W // CHUNK
NPAIR = NCHUNK // 2

_mesh = plsc.VectorSubcoreMesh(core_axis_name="c", subcore_axis_name="s")


@functools.partial(
    pl.kernel,
    out_type=jax.ShapeDtypeStruct((TOTAL, HIDDEN), jnp.float32),
    mesh=_mesh,
    scratch_types=[
        pltpu.VMEM((4, CHUNK), jnp.int32),
        pltpu.VMEM((4, CHUNK, HIDDEN), jnp.float32),
        pltpu.SemaphoreType.DMA((4,)),
        pltpu.SemaphoreType.DMA((2,)),
        pltpu.SemaphoreType.DMA((4,)),
    ],
    compiler_params=pltpu.CompilerParams(use_tc_tiling_on_sc=False),
)
def _emb_lookup(x_hbm, tab_hbm, out_hbm, idx_v, rows_v, isem, gsem, osem):
    wid = lax.axis_index("s") * NUM_CORES + lax.axis_index("c")
    base = wid * PER_W

    def xs(j):
        return x_hbm.at[pl.ds(base + j * CHUNK, CHUNK)]

    def outs(j):
        return out_hbm.at[pl.ds(base + j * CHUNK, CHUNK)]

    def istart(j, b):
        pltpu.async_copy(xs(j), idx_v.at[b], isem.at[b])

    def iwait(j, b):
        pltpu.make_async_copy(xs(j), idx_v.at[b], isem.at[b]).wait()

    def gstart(b, e):
        pltpu.async_copy(tab_hbm.at[idx_v.at[b]], rows_v.at[b], gsem.at[e])

    def gwait(b, e):
        pltpu.make_async_copy(tab_hbm.at[idx_v.at[b]], rows_v.at[b],
                              gsem.at[e]).wait()

    def ostart(j, b):
        pltpu.async_copy(rows_v.at[b], outs(j), osem.at[b])

    def owait(j, b):
        pltpu.make_async_copy(rows_v.at[b], outs(j), osem.at[b]).wait()

    def fire(p, e, wait_rows):
        j0 = 2 * p
        s0, s1 = 2 * e, 2 * e + 1
        iwait(j0, s0)
        iwait(j0 + 1, s1)
        if wait_rows:
            owait(j0 - 4, s0)
            owait(j0 - 3, s1)
        gstart(s0, e)
        gstart(s1, e)

    def drain(p, e, refill):
        j0 = 2 * p
        s0, s1 = 2 * e, 2 * e + 1
        gwait(s0, e)
        gwait(s1, e)
        ostart(j0, s0)
        ostart(j0 + 1, s1)
        if refill:
            istart(j0 + 4, s0)
            istart(j0 + 5, s1)

    for b in range(4):
        istart(b, b)
    fire(0, 0, False)
    fire(1, 1, False)

    def body(k, carry):
        p = 2 * k
        drain(p, 0, True)
        fire(p + 2, 0, True)
        drain(p + 1, 1, True)
        fire(p + 3, 1, True)
        return carry

    lax.fori_loop(0, (NPAIR - 2) // 2, body, 0)

    drain(NPAIR - 2, 0, False)
    drain(NPAIR - 1, 1, False)
    for b in range(4):
        owait(NCHUNK - 4 + b, b)


def kernel(x, table):
    flat = x.reshape(TOTAL)
    out = _emb_lookup(flat, table)
    return out.reshape(BATCH, HIST, HIDDEN)
